# baseline (speedup 1.0000x reference)
import jax
import jax.numpy as jnp
from jax import lax
from jax.experimental import pallas as pl
from jax.experimental.pallas import tpu as pltpu

_TOPK = 100
_THRESH = -1000000000.0
_NEG = -1000000000.0


def _halve_max(cur, w):
    while w > 1:
        w //= 2
        cur = jnp.maximum(cur[:, :w, :], cur[:, w:, :])
    return cur


def _halve_or(cur, w):
    while w > 1:
        w //= 2
        cur = cur[:, :w, :] | cur[:, w:, :]
    return cur


def _sizes(R):
    P = R // 2
    G1 = 16 if P % 16 == 0 else P
    N1 = P // G1
    C = 512 if R % 512 == 0 else R
    NC = R // C
    return P, G1, N1, C, NC


def _make_body(BS, R, W, H, topk):
    P, G1, N1, C, NC = _sizes(R)
    PC = C // 2
    LC = PC // G1

    def body(x_ref, top_ref, bot_ref, score_ref, view_ref, row_ref, col_ref,
             *scratch):
        vs = scratch[0::3]
        pas = scratch[1::3]
        l1s = scratch[2::3]
        step = pl.program_id(0)
        b = step // NC
        k = step % NC
        ninf = jnp.float32(-jnp.inf)

        @pl.when(step < BS * NC)
        def nms_phase():
            nrow2 = jnp.full((2, W), ninf, jnp.float32)
            top2 = jnp.where(k > 0, top_ref[0, 6:8, :], nrow2)
            bot2 = jnp.where(k < NC - 1, bot_ref[0, 0:2, :], nrow2)
            xa = jnp.concatenate([top2, x_ref[0], bot2], 0)
            ncol1 = jnp.full((C + 4, 1), ninf, jnp.float32)
            ncol2 = jnp.full((C + 4, 2), ninf, jnp.float32)
            h = jnp.maximum(
                jnp.maximum(xa, jnp.concatenate([xa[:, 1:], ncol1], 1)),
                jnp.concatenate([ncol1, xa[:, :-1]], 1),
            )
            h = jnp.maximum(
                h,
                jnp.maximum(
                    jnp.concatenate([xa[:, 2:], ncol2], 1),
                    jnp.concatenate([ncol2, xa[:, :-2]], 1),
                ),
            )
            vv = jnp.maximum(
                jnp.maximum(h[2: C + 2, :], h[: C, :]),
                jnp.maximum(h[1: C + 1, :], h[3: C + 3, :]),
            )
            vv = jnp.maximum(vv, h[4: C + 4, :])
            xc = xa[2: C + 2, :]
            m = jnp.where(xc == vv, xc, jnp.float32(_NEG))
            nrow1 = jnp.full((1, W), ninf, jnp.float32)
            sdown = jnp.concatenate([m[1:, :], nrow1], 0)
            win_full = jnp.maximum(m, sdown)
            paf = jnp.where(sdown > m, 1, jnp.where(sdown == m, 2, 0))
            rowi = lax.broadcasted_iota(jnp.int32, (C, W), 0)
            even = (rowi & 1) == 0
            rr2 = (rowi >> 1) & 15
            contrib = jnp.where(even, paf << (2 * rr2), 0)
            packed = _halve_or(
                contrib.reshape(PC // 16, 32, W), 32).reshape(PC // 16, W)
            win = win_full.reshape(PC, 2, W)[:, 0, :]
            l1c = _halve_max(
                jnp.where(even, win_full, ninf).reshape(LC, 2 * G1, W), 2 * G1
            ).reshape(LC, W)
            for bb in range(BS):
                @pl.when(b == bb)
                def store_chunk(bb=bb):
                    vs[bb][pl.ds(k * PC, PC), :] = win
                    pas[bb][pl.ds(k * (PC // 16), PC // 16), :] = packed
                    l1s[bb][pl.ds(k * LC, LC), :] = l1c

        @pl.when(step == BS * NC)
        def select_phase():
            lane128 = lax.broadcasted_iota(jnp.int32, (1, 128), 1)
            iotan1 = lax.broadcasted_iota(jnp.int32, (N1, W), 0)
            iotag1 = lax.broadcasted_iota(jnp.int32, (G1, W), 0)
            iotac = lax.broadcasted_iota(jnp.int32, (1, W), 1)
            brow = lax.broadcasted_iota(jnp.int32, (BS, 128), 0)

            def iter_body(i, carry):
                sc, vw, rw, cw = carry
                lmr = lane128 == i
                for bb in range(BS):
                    v_ref, pa_ref, l1_ref = vs[bb], pas[bb], l1s[bb]
                    l1 = l1_ref[...]
                    vb = jnp.max(
                        jnp.max(l1, axis=0, keepdims=True), axis=1, keepdims=True)
                    s1 = jnp.min(jnp.where(l1 == vb, iotan1, N1))
                    vg = v_ref[pl.ds(s1 * G1, G1), :]
                    prow = pa_ref[pl.ds(s1, 1), :]
                    eq16 = vg == vb
                    s2l = jnp.min(
                        jnp.min(jnp.where(eq16, iotag1, G1), axis=0,
                                keepdims=True), axis=1, keepdims=True)
                    rm16 = iotag1 == s2l
                    vrow = jnp.max(
                        jnp.where(rm16, vg, ninf), axis=0, keepdims=True)
                    shv = 2 * s2l
                    pav = (prow >> shv) & 3
                    peff = (pav == 1).astype(jnp.int32)
                    eq = vrow == vb
                    minp = jnp.min(jnp.where(eq, peff, 2), axis=1, keepdims=True)
                    sel = eq & (peff == minp)
                    c = jnp.min(jnp.where(sel, iotac, W), axis=1, keepdims=True)
                    lc = iotac == c
                    vg_new = jnp.where(rm16 & lc & (pav != 2), ninf, vg)
                    v_ref[pl.ds(s1 * G1, G1), :] = vg_new
                    pa_ref[pl.ds(s1, 1), :] = jnp.where(
                        lc, (prow & ~(3 << shv)) | (1 << shv), prow)
                    l1_ref[pl.ds(s1, 1), :] = jnp.max(
                        vg_new, axis=0, keepdims=True)
                    upd = (brow == bb) & lmr
                    s2v = s1 * G1 + s2l
                    sc = jnp.where(upd, vb, sc)
                    vw = jnp.where(upd, (s2v // (H // 2)).astype(jnp.float32), vw)
                    rw = jnp.where(
                        upd, ((2 * s2v) % H + minp).astype(jnp.float32), rw)
                    cw = jnp.where(upd, c.astype(jnp.float32), cw)
                return sc, vw, rw, cw

            zero = jnp.zeros((BS, 128), jnp.float32)
            sc, vw, rw, cw = lax.fori_loop(
                0, topk, iter_body, (zero, zero, zero, zero))
            score_ref[...] = sc
            view_ref[...] = vw
            row_ref[...] = rw
            col_ref[...] = cw

    return body


def kernel(heatmap_logits):
    bs, num_img, _, H, W = heatmap_logits.shape
    R = num_img * H
    hm = heatmap_logits.reshape(bs, R, W)
    topk = min(_TOPK, R * W)
    P, _, N1, C, NC = _sizes(R)
    C8 = C // 8
    R8 = R // 8
    S = bs * NC

    def ix_main(s):
        bb = jnp.minimum(s // NC, bs - 1)
        return (bb, jnp.where(s < S, s % NC, 0), 0)

    def ix_top(s):
        bb = jnp.minimum(s // NC, bs - 1)
        return (bb, jnp.maximum((s % NC) * C8 - 1, 0), 0)

    def ix_bot(s):
        bb = jnp.minimum(s // NC, bs - 1)
        return (bb, jnp.minimum((s % NC) * C8 + C8, R8 - 1), 0)

    scratch = []
    for _ in range(bs):
        scratch += [
            pltpu.VMEM((P, W), jnp.float32),
            pltpu.VMEM((P // 16, W), jnp.int32),
            pltpu.VMEM((N1, W), jnp.float32),
        ]

    body = _make_body(bs, R, W, H, topk)
    outs = pl.pallas_call(
        body,
        grid=(S + 1,),
        in_specs=[
            pl.BlockSpec((1, C, W), ix_main),
            pl.BlockSpec((1, 8, W), ix_top),
            pl.BlockSpec((1, 8, W), ix_bot),
        ],
        out_specs=[pl.BlockSpec((bs, 128), lambda s: (0, 0)) for _ in range(4)],
        out_shape=[jax.ShapeDtypeStruct((bs, 128), jnp.float32) for _ in range(4)],
        scratch_shapes=scratch,
    )(hm, hm, hm)
    scores128, views128, rows128, cols128 = outs
    scores = scores128[:, :topk]
    peak_positions = jnp.stack(
        [views128[:, :topk], rows128[:, :topk], cols128[:, :topk]], axis=-1
    )
    peak_mask = scores > _THRESH
    return peak_positions, scores, peak_mask


# speedup vs baseline: 1.2035x; 1.2035x over previous
import jax
import jax.numpy as jnp
from jax import lax
from jax.experimental import pallas as pl
from jax.experimental.pallas import tpu as pltpu

_TOPK = 100
_THRESH = -1000000000.0
_NEG = -1000000000.0


def _halve_max(cur, w):
    while w > 1:
        w //= 2
        cur = jnp.maximum(cur[:, :w, :], cur[:, w:, :])
    return cur


def _halve_or(cur, w):
    while w > 1:
        w //= 2
        cur = cur[:, :w, :] | cur[:, w:, :]
    return cur


def _sizes(R):
    P = R // 2
    G1 = 16 if P % 16 == 0 else P
    N1 = P // G1
    C = 512 if R % 512 == 0 else R
    NC = R // C
    return P, G1, N1, C, NC


def _make_body(BS, R, W, H, topk):
    P, G1, N1, C, NC = _sizes(R)
    PC = C // 2
    LC = PC // G1

    def body(x_ref, top_ref, bot_ref, score_ref, view_ref, row_ref, col_ref,
             *scratch):
        vs = scratch[0::3]
        pas = scratch[1::3]
        l1s = scratch[2::3]
        step = pl.program_id(0)
        b = step // NC
        k = step % NC
        ninf = jnp.float32(-jnp.inf)

        def horiz5(z):
            n = z.shape[0]
            nc1 = jnp.full((n, 1), ninf, jnp.float32)
            nc2 = jnp.full((n, 2), ninf, jnp.float32)
            a = jnp.maximum(z, jnp.concatenate([z[:, 1:], nc1], 1))
            b = jnp.maximum(a, jnp.concatenate([nc2, a[:, :-2]], 1))
            return jnp.maximum(b, jnp.concatenate([z[:, 2:], nc2], 1))

        @pl.when(step < BS * NC)
        def nms_phase():
            nrow2 = jnp.full((2, W), ninf, jnp.float32)
            top2 = jnp.where(k > 0, top_ref[0, 6:8, :], nrow2)
            bot2 = jnp.where(k < NC - 1, bot_ref[0, 0:2, :], nrow2)
            xc = x_ref[0]
            ha = jnp.concatenate(
                [horiz5(top2), horiz5(xc), horiz5(bot2)], 0)
            av = jnp.maximum(ha[:-1, :], ha[1:, :])
            vv = jnp.maximum(
                jnp.maximum(av[: C, :], av[2: C + 2, :]), ha[4: C + 4, :])
            m = jnp.where(xc == vv, xc, jnp.float32(_NEG))
            m2 = m.reshape(PC, 2, W)
            r0 = m2[:, 0, :]
            r1 = m2[:, 1, :]
            win = jnp.maximum(r0, r1)
            pa = jnp.where(r1 > r0, 1, jnp.where(r1 == r0, 2, 0))
            rr = lax.broadcasted_iota(jnp.int32, (PC, W), 0) % 16
            packed = _halve_or((pa << (2 * rr)).reshape(PC // 16, 16, W), 16)
            packed = packed.reshape(PC // 16, W)
            l1c = _halve_max(win.reshape(LC, G1, W), G1).reshape(LC, W)
            for bb in range(BS):
                @pl.when(b == bb)
                def store_chunk(bb=bb):
                    vs[bb][pl.ds(k * PC, PC), :] = win
                    pas[bb][pl.ds(k * (PC // 16), PC // 16), :] = packed
                    l1s[bb][pl.ds(k * LC, LC), :] = l1c

        @pl.when(step == BS * NC)
        def select_phase():
            lane128 = lax.broadcasted_iota(jnp.int32, (1, 128), 1)
            iotan1 = lax.broadcasted_iota(jnp.int32, (N1, W), 0)
            iotag1 = lax.broadcasted_iota(jnp.int32, (G1, W), 0)
            iotac = lax.broadcasted_iota(jnp.int32, (1, W), 1)
            brow = lax.broadcasted_iota(jnp.int32, (BS, 128), 0)

            def iter_body(i, carry):
                sc, vw, rw, cw = carry
                lmr = lane128 == i
                for bb in range(BS):
                    v_ref, pa_ref, l1_ref = vs[bb], pas[bb], l1s[bb]
                    l1 = l1_ref[...]
                    vb = jnp.max(
                        jnp.max(l1, axis=0, keepdims=True), axis=1, keepdims=True)
                    s1 = jnp.min(jnp.where(l1 == vb, iotan1, N1))
                    vg = v_ref[pl.ds(s1 * G1, G1), :]
                    prow = pa_ref[pl.ds(s1, 1), :]
                    eq16 = vg == vb
                    s2l = jnp.min(
                        jnp.min(jnp.where(eq16, iotag1, G1), axis=0,
                                keepdims=True), axis=1, keepdims=True)
                    rm16 = iotag1 == s2l
                    vrow = jnp.max(
                        jnp.where(rm16, vg, ninf), axis=0, keepdims=True)
                    shv = 2 * s2l
                    pav = (prow >> shv) & 3
                    peff = (pav == 1).astype(jnp.int32)
                    eq = vrow == vb
                    minp = jnp.min(jnp.where(eq, peff, 2), axis=1, keepdims=True)
                    sel = eq & (peff == minp)
                    c = jnp.min(jnp.where(sel, iotac, W), axis=1, keepdims=True)
                    lc = iotac == c
                    vg_new = jnp.where(rm16 & lc & (pav != 2), ninf, vg)
                    v_ref[pl.ds(s1 * G1, G1), :] = vg_new
                    pa_ref[pl.ds(s1, 1), :] = jnp.where(
                        lc, (prow & ~(3 << shv)) | (1 << shv), prow)
                    l1_ref[pl.ds(s1, 1), :] = jnp.max(
                        vg_new, axis=0, keepdims=True)
                    upd = (brow == bb) & lmr
                    s2v = s1 * G1 + s2l
                    sc = jnp.where(upd, vb, sc)
                    vw = jnp.where(upd, (s2v // (H // 2)).astype(jnp.float32), vw)
                    rw = jnp.where(
                        upd, ((2 * s2v) % H + minp).astype(jnp.float32), rw)
                    cw = jnp.where(upd, c.astype(jnp.float32), cw)
                return sc, vw, rw, cw

            zero = jnp.zeros((BS, 128), jnp.float32)
            sc, vw, rw, cw = lax.fori_loop(
                0, topk, iter_body, (zero, zero, zero, zero))
            score_ref[...] = sc
            view_ref[...] = vw
            row_ref[...] = rw
            col_ref[...] = cw

    return body


def kernel(heatmap_logits):
    bs, num_img, _, H, W = heatmap_logits.shape
    R = num_img * H
    hm = heatmap_logits.reshape(bs, R, W)
    topk = min(_TOPK, R * W)
    P, _, N1, C, NC = _sizes(R)
    C8 = C // 8
    R8 = R // 8
    S = bs * NC

    def ix_main(s):
        bb = jnp.minimum(s // NC, bs - 1)
        return (bb, jnp.where(s < S, s % NC, 0), 0)

    def ix_top(s):
        bb = jnp.minimum(s // NC, bs - 1)
        return (bb, jnp.maximum((s % NC) * C8 - 1, 0), 0)

    def ix_bot(s):
        bb = jnp.minimum(s // NC, bs - 1)
        return (bb, jnp.minimum((s % NC) * C8 + C8, R8 - 1), 0)

    scratch = []
    for _ in range(bs):
        scratch += [
            pltpu.VMEM((P, W), jnp.float32),
            pltpu.VMEM((P // 16, W), jnp.int32),
            pltpu.VMEM((N1, W), jnp.float32),
        ]

    body = _make_body(bs, R, W, H, topk)
    outs = pl.pallas_call(
        body,
        grid=(S + 1,),
        in_specs=[
            pl.BlockSpec((1, C, W), ix_main),
            pl.BlockSpec((1, 8, W), ix_top),
            pl.BlockSpec((1, 8, W), ix_bot),
        ],
        out_specs=[pl.BlockSpec((bs, 128), lambda s: (0, 0)) for _ in range(4)],
        out_shape=[jax.ShapeDtypeStruct((bs, 128), jnp.float32) for _ in range(4)],
        scratch_shapes=scratch,
    )(hm, hm, hm)
    scores128, views128, rows128, cols128 = outs
    scores = scores128[:, :topk]
    peak_positions = jnp.stack(
        [views128[:, :topk], rows128[:, :topk], cols128[:, :topk]], axis=-1
    )
    peak_mask = scores > _THRESH
    return peak_positions, scores, peak_mask


# speedup vs baseline: 1.2274x; 1.0199x over previous
import jax
import jax.numpy as jnp
from jax import lax
from jax.experimental import pallas as pl
from jax.experimental.pallas import tpu as pltpu

_TOPK = 100
_THRESH = -1000000000.0
_NEG = -1000000000.0


def _halve_max(cur, w):
    while w > 1:
        w //= 2
        cur = jnp.maximum(cur[:, :w, :], cur[:, w:, :])
    return cur


def _halve_or(cur, w):
    while w > 1:
        w //= 2
        cur = cur[:, :w, :] | cur[:, w:, :]
    return cur


def _sizes(R):
    P = R // 2
    G1 = 16 if P % 16 == 0 else P
    N1 = P // G1
    C = 1024 if R % 1024 == 0 else (512 if R % 512 == 0 else R)
    NC = R // C
    return P, G1, N1, C, NC


def _make_body(BS, R, W, H, topk):
    P, G1, N1, C, NC = _sizes(R)
    PC = C // 2
    LC = PC // G1

    def body(x_ref, top_ref, bot_ref, score_ref, view_ref, row_ref, col_ref,
             *scratch):
        vs = scratch[0::3]
        pas = scratch[1::3]
        l1s = scratch[2::3]
        step = pl.program_id(0)
        b = step // NC
        k = step % NC
        ninf = jnp.float32(-jnp.inf)

        def horiz5(z):
            n = z.shape[0]
            nc1 = jnp.full((n, 1), ninf, jnp.float32)
            nc2 = jnp.full((n, 2), ninf, jnp.float32)
            a = jnp.maximum(z, jnp.concatenate([z[:, 1:], nc1], 1))
            b = jnp.maximum(a, jnp.concatenate([nc2, a[:, :-2]], 1))
            return jnp.maximum(b, jnp.concatenate([z[:, 2:], nc2], 1))

        @pl.when(step < BS * NC)
        def nms_phase():
            nrow2 = jnp.full((2, W), ninf, jnp.float32)
            top2 = jnp.where(k > 0, top_ref[0, 6:8, :], nrow2)
            bot2 = jnp.where(k < NC - 1, bot_ref[0, 0:2, :], nrow2)
            xc = x_ref[0]
            ha = jnp.concatenate(
                [horiz5(top2), horiz5(xc), horiz5(bot2)], 0)
            av = jnp.maximum(ha[:-1, :], ha[1:, :])
            vv = jnp.maximum(
                jnp.maximum(av[: C, :], av[2: C + 2, :]), ha[4: C + 4, :])
            m = jnp.where(xc == vv, xc, jnp.float32(_NEG))
            m2 = m.reshape(PC, 2, W)
            r0 = m2[:, 0, :]
            r1 = m2[:, 1, :]
            win = jnp.maximum(r0, r1)
            pa = jnp.where(r1 > r0, 1, jnp.where(r1 == r0, 2, 0))
            rr = lax.broadcasted_iota(jnp.int32, (PC, W), 0) % 16
            packed = _halve_or((pa << (2 * rr)).reshape(PC // 16, 16, W), 16)
            packed = packed.reshape(PC // 16, W)
            l1c = _halve_max(win.reshape(LC, G1, W), G1).reshape(LC, W)
            for bb in range(BS):
                @pl.when(b == bb)
                def store_chunk(bb=bb):
                    vs[bb][pl.ds(k * PC, PC), :] = win
                    pas[bb][pl.ds(k * (PC // 16), PC // 16), :] = packed
                    l1s[bb][pl.ds(k * LC, LC), :] = l1c

        @pl.when(step == BS * NC)
        def select_phase():
            lane128 = lax.broadcasted_iota(jnp.int32, (1, 128), 1)
            iotan1 = lax.broadcasted_iota(jnp.int32, (N1, W), 0)
            iotag1 = lax.broadcasted_iota(jnp.int32, (G1, W), 0)
            iotac = lax.broadcasted_iota(jnp.int32, (1, W), 1)
            brow = lax.broadcasted_iota(jnp.int32, (BS, 128), 0)

            def iter_body(i, carry):
                sc, vw, rw, cw = carry
                lmr = lane128 == i
                for bb in range(BS):
                    v_ref, pa_ref, l1_ref = vs[bb], pas[bb], l1s[bb]
                    l1 = l1_ref[...]
                    vb = jnp.max(
                        jnp.max(l1, axis=0, keepdims=True), axis=1, keepdims=True)
                    s1 = jnp.min(jnp.where(l1 == vb, iotan1, N1))
                    vg = v_ref[pl.ds(s1 * G1, G1), :]
                    prow = pa_ref[pl.ds(s1, 1), :]
                    eq16 = vg == vb
                    s2l = jnp.min(
                        jnp.min(jnp.where(eq16, iotag1, G1), axis=0,
                                keepdims=True), axis=1, keepdims=True)
                    rm16 = iotag1 == s2l
                    vrow = jnp.max(
                        jnp.where(rm16, vg, ninf), axis=0, keepdims=True)
                    shv = 2 * s2l
                    pav = (prow >> shv) & 3
                    peff = (pav == 1).astype(jnp.int32)
                    eq = vrow == vb
                    minp = jnp.min(jnp.where(eq, peff, 2), axis=1, keepdims=True)
                    sel = eq & (peff == minp)
                    c = jnp.min(jnp.where(sel, iotac, W), axis=1, keepdims=True)
                    lc = iotac == c
                    vg_new = jnp.where(rm16 & lc & (pav != 2), ninf, vg)
                    v_ref[pl.ds(s1 * G1, G1), :] = vg_new
                    pa_ref[pl.ds(s1, 1), :] = jnp.where(
                        lc, (prow & ~(3 << shv)) | (1 << shv), prow)
                    l1_ref[pl.ds(s1, 1), :] = jnp.max(
                        vg_new, axis=0, keepdims=True)
                    upd = (brow == bb) & lmr
                    s2v = s1 * G1 + s2l
                    sc = jnp.where(upd, vb, sc)
                    vw = jnp.where(upd, (s2v // (H // 2)).astype(jnp.float32), vw)
                    rw = jnp.where(
                        upd, ((2 * s2v) % H + minp).astype(jnp.float32), rw)
                    cw = jnp.where(upd, c.astype(jnp.float32), cw)
                return sc, vw, rw, cw

            zero = jnp.zeros((BS, 128), jnp.float32)
            sc, vw, rw, cw = lax.fori_loop(
                0, topk, iter_body, (zero, zero, zero, zero))
            score_ref[...] = sc
            view_ref[...] = vw
            row_ref[...] = rw
            col_ref[...] = cw

    return body


def kernel(heatmap_logits):
    bs, num_img, _, H, W = heatmap_logits.shape
    R = num_img * H
    hm = heatmap_logits.reshape(bs, R, W)
    topk = min(_TOPK, R * W)
    P, _, N1, C, NC = _sizes(R)
    C8 = C // 8
    R8 = R // 8
    S = bs * NC

    def ix_main(s):
        bb = jnp.minimum(s // NC, bs - 1)
        return (bb, jnp.where(s < S, s % NC, 0), 0)

    def ix_top(s):
        bb = jnp.minimum(s // NC, bs - 1)
        return (bb, jnp.maximum((s % NC) * C8 - 1, 0), 0)

    def ix_bot(s):
        bb = jnp.minimum(s // NC, bs - 1)
        return (bb, jnp.minimum((s % NC) * C8 + C8, R8 - 1), 0)

    scratch = []
    for _ in range(bs):
        scratch += [
            pltpu.VMEM((P, W), jnp.float32),
            pltpu.VMEM((P // 16, W), jnp.int32),
            pltpu.VMEM((N1, W), jnp.float32),
        ]

    body = _make_body(bs, R, W, H, topk)
    outs = pl.pallas_call(
        body,
        grid=(S + 1,),
        in_specs=[
            pl.BlockSpec((1, C, W), ix_main),
            pl.BlockSpec((1, 8, W), ix_top),
            pl.BlockSpec((1, 8, W), ix_bot),
        ],
        out_specs=[pl.BlockSpec((bs, 128), lambda s: (0, 0)) for _ in range(4)],
        out_shape=[jax.ShapeDtypeStruct((bs, 128), jnp.float32) for _ in range(4)],
        scratch_shapes=scratch,
    )(hm, hm, hm)
    scores128, views128, rows128, cols128 = outs
    scores = scores128[:, :topk]
    peak_positions = jnp.stack(
        [views128[:, :topk], rows128[:, :topk], cols128[:, :topk]], axis=-1
    )
    peak_mask = scores > _THRESH
    return peak_positions, scores, peak_mask
